# baseline (speedup 1.0000x reference)
import functools

import jax
import jax.numpy as jnp
from jax import lax
from jax.experimental import pallas as pl
from jax.experimental.pallas import tpu as pltpu
from jax.experimental.pallas import tpu_sc as plsc

NC, NS = 2, 16
NW = NC * NS
D = 512
LANES = 16
B_TOTAL = 4096 * 200
B_PER_W = B_TOTAL // NW
CHUNK = 32
NCHUNKS = B_PER_W // CHUNK
NHALF = NCHUNKS // 2
NBUF = 2
VOCAB_PAD = 64


def _bcast_lane(vec, lane):
    return lax.gather(
        vec,
        jnp.full((LANES, 1), lane, jnp.int32),
        lax.GatherDimensionNumbers(
            offset_dims=(), collapsed_slice_dims=(0,), start_index_map=(0,)),
        (1,),
        mode=lax.GatherScatterMode.PROMISE_IN_BOUNDS,
    )


def _sc_body(idx_hbm, table_hbm, out_hbm,
             idx_v, table_v, buf0, buf1, ssem0, ssem1):
    wid = lax.axis_index("s") * NC + lax.axis_index("c")
    base_row = wid * B_PER_W

    pltpu.sync_copy(table_hbm, table_v)

    bufs = (buf0, buf1)
    ssems = (ssem0, ssem1)
    col0 = lax.iota(jnp.int32, LANES)

    def out_rows(jg):
        return out_hbm.at[pl.ds(base_row + jg * CHUNK, CHUNK)]

    def gen_chunk(j, buf):
        def group(g16, _):
            idx_vec = idx_v[j, pl.ds(g16 * LANES, LANES)]
            for l in range(LANES):
                base = _bcast_lane(idx_vec, l) * D + col0
                r = g16 * LANES + l
                for k in range(D // LANES):
                    v = plsc.load_gather(table_v, [base + (k * LANES)])
                    buf[r, pl.ds(k * LANES, LANES)] = v
            return ()
        lax.fori_loop(0, CHUNK // LANES, group, (), unroll=False)

    for half in range(2):
        pltpu.sync_copy(idx_hbm.at[wid, half], idx_v)

        def step(g, _, half=half):
            for b in range(NBUF):
                j = g * NBUF + b
                jg = half * NHALF + j

                if half == 0:
                    @pl.when(g > 0)
                    def _():
                        pltpu.make_async_copy(
                            bufs[b], out_rows(jg - NBUF), ssems[b]).wait()
                else:
                    pltpu.make_async_copy(
                        bufs[b], out_rows(jg - NBUF), ssems[b]).wait()

                gen_chunk(j, bufs[b])
                pltpu.async_copy(bufs[b], out_rows(jg), ssems[b])
            return ()

        lax.fori_loop(0, NHALF // NBUF, step, (), unroll=False)

    for b in range(NBUF):
        jg = NCHUNKS - NBUF + b
        pltpu.make_async_copy(bufs[b], out_rows(jg), ssems[b]).wait()


@jax.jit
def _lookup(idx, W_flat):
    mesh = plsc.VectorSubcoreMesh(core_axis_name="c", subcore_axis_name="s")
    f = pl.kernel(
        _sc_body,
        out_type=jax.ShapeDtypeStruct((B_TOTAL, D), jnp.float32),
        mesh=mesh,
        compiler_params=pltpu.CompilerParams(needs_layout_passes=False),
        scratch_types=[
            pltpu.VMEM((NHALF, CHUNK), jnp.int32),
            pltpu.VMEM((VOCAB_PAD * D,), jnp.float32),
            pltpu.VMEM((CHUNK, D), jnp.float32),
            pltpu.VMEM((CHUNK, D), jnp.float32),
            pltpu.SemaphoreType.DMA,
            pltpu.SemaphoreType.DMA,
        ],
    )
    return f(idx, W_flat)


def kernel(x, W):
    idx = x[:, :, 1].astype(jnp.int32).reshape(NW, 2, NHALF, CHUNK)
    W_pad = jnp.pad(W, ((0, VOCAB_PAD - W.shape[0]), (0, 0)))
    out = _lookup(idx, W_pad.reshape(VOCAB_PAD * D))
    return out.reshape(4096, 200, D)


# speedup vs baseline: 8.5793x; 8.5793x over previous
import functools

import jax
import jax.numpy as jnp
from jax import lax
from jax.experimental import pallas as pl
from jax.experimental.pallas import tpu as pltpu
from jax.experimental.pallas import tpu_sc as plsc

NC, NS = 2, 16
NW = NC * NS
D = 512
LANES = 16
B_TOTAL = 4096 * 200
B_PER_W = B_TOTAL // NW
CHUNK = 32
NCHUNKS = B_PER_W // CHUNK
NHALF = NCHUNKS // 2
NBUF = 2
VOCAB_PAD = 64


def _bcast_lane(vec, lane):
    return lax.gather(
        vec,
        jnp.full((LANES, 1), lane, jnp.int32),
        lax.GatherDimensionNumbers(
            offset_dims=(), collapsed_slice_dims=(0,), start_index_map=(0,)),
        (1,),
        mode=lax.GatherScatterMode.PROMISE_IN_BOUNDS,
    )


def _sc_body(idx_hbm, table_hbm, out_hbm,
             idx_v, table_v, buf0, buf1, ssem0, ssem1):
    wid = lax.axis_index("s") * NC + lax.axis_index("c")
    base_row = wid * B_PER_W

    pltpu.sync_copy(table_hbm, table_v)

    bufs = (buf0, buf1)
    ssems = (ssem0, ssem1)
    col0 = lax.iota(jnp.int32, LANES)

    def out_rows(jg):
        return out_hbm.at[pl.ds(base_row + jg * CHUNK, CHUNK)]

    def gen_chunk(j, buf):
        @plsc.parallel_loop(0, CHUNK, 1, unroll=2)
        def row_body(r):
            g16 = r // LANES
            lane = r % LANES
            idx_vec = idx_v[j, pl.ds(g16 * LANES, LANES)]
            base = _bcast_lane(idx_vec, lane) * D + col0
            for k in range(D // LANES):
                v = plsc.load_gather(table_v, [base + (k * LANES)])
                buf[r, pl.ds(k * LANES, LANES)] = v

    for half in range(2):
        pltpu.sync_copy(idx_hbm.at[wid, half], idx_v)

        def step(g, _, half=half):
            for b in range(NBUF):
                j = g * NBUF + b
                jg = half * NHALF + j

                if half == 0:
                    @pl.when(g > 0)
                    def _():
                        pltpu.make_async_copy(
                            bufs[b], out_rows(jg - NBUF), ssems[b]).wait()
                else:
                    pltpu.make_async_copy(
                        bufs[b], out_rows(jg - NBUF), ssems[b]).wait()

                gen_chunk(j, bufs[b])
                pltpu.async_copy(bufs[b], out_rows(jg), ssems[b])
            return ()

        lax.fori_loop(0, NHALF // NBUF, step, (), unroll=False)

    for b in range(NBUF):
        jg = NCHUNKS - NBUF + b
        pltpu.make_async_copy(bufs[b], out_rows(jg), ssems[b]).wait()


@jax.jit
def _lookup(idx, W_flat):
    mesh = plsc.VectorSubcoreMesh(core_axis_name="c", subcore_axis_name="s")
    f = pl.kernel(
        _sc_body,
        out_type=jax.ShapeDtypeStruct((B_TOTAL, D), jnp.float32),
        mesh=mesh,
        compiler_params=pltpu.CompilerParams(needs_layout_passes=False),
        scratch_types=[
            pltpu.VMEM((NHALF, CHUNK), jnp.int32),
            pltpu.VMEM((VOCAB_PAD * D,), jnp.float32),
            pltpu.VMEM((CHUNK, D), jnp.float32),
            pltpu.VMEM((CHUNK, D), jnp.float32),
            pltpu.SemaphoreType.DMA,
            pltpu.SemaphoreType.DMA,
        ],
    )
    return f(idx, W_flat)


def kernel(x, W):
    idx = x[:, :, 1].astype(jnp.int32).reshape(NW, 2, NHALF, CHUNK)
    W_pad = jnp.pad(W, ((0, VOCAB_PAD - W.shape[0]), (0, 0)))
    out = _lookup(idx, W_pad.reshape(VOCAB_PAD * D))
    return out.reshape(4096, 200, D)


# speedup vs baseline: 8.9418x; 1.0423x over previous
import functools

import jax
import jax.numpy as jnp
from jax import lax
from jax.experimental import pallas as pl
from jax.experimental.pallas import tpu as pltpu
from jax.experimental.pallas import tpu_sc as plsc

NC, NS = 2, 16
NW = NC * NS
D = 512
LANES = 16
B_TOTAL = 4096 * 200
B_PER_W = B_TOTAL // NW
CHUNK = 64
NCHUNKS = B_PER_W // CHUNK
NSTAGE = 4
NPART = NCHUNKS // NSTAGE
NBUF = 2
VOCAB_PAD = 64


def _bcast_lane(vec, lane):
    return lax.gather(
        vec,
        jnp.full((LANES, 1), lane, jnp.int32),
        lax.GatherDimensionNumbers(
            offset_dims=(), collapsed_slice_dims=(0,), start_index_map=(0,)),
        (1,),
        mode=lax.GatherScatterMode.PROMISE_IN_BOUNDS,
    )


def _sc_body(idx_hbm, table_hbm, out_hbm,
             idx_v, table_v, buf0, buf1, ssem0, ssem1):
    wid = lax.axis_index("s") * NC + lax.axis_index("c")
    base_row = wid * B_PER_W

    pltpu.sync_copy(table_hbm, table_v)

    bufs = (buf0, buf1)
    ssems = (ssem0, ssem1)
    col0 = lax.iota(jnp.int32, LANES)

    def out_rows(jg):
        return out_hbm.at[pl.ds(base_row + jg * CHUNK, CHUNK)]

    def gen_chunk(j, buf):
        @plsc.parallel_loop(0, CHUNK, 1, unroll=2)
        def row_body(r):
            g16 = r // LANES
            lane = r % LANES
            idx_vec = idx_v[j, pl.ds(g16 * LANES, LANES)]
            base = _bcast_lane(idx_vec, lane) * D + col0
            for k in range(D // LANES):
                v = plsc.load_gather(table_v, [base + (k * LANES)])
                buf[r, pl.ds(k * LANES, LANES)] = v

    for part in range(NSTAGE):
        pltpu.sync_copy(idx_hbm.at[wid, part], idx_v)

        def step(g, _, part=part):
            for b in range(NBUF):
                j = g * NBUF + b
                jg = part * NPART + j

                if part == 0:
                    @pl.when(g > 0)
                    def _():
                        pltpu.make_async_copy(
                            bufs[b], out_rows(jg - NBUF), ssems[b]).wait()
                else:
                    pltpu.make_async_copy(
                        bufs[b], out_rows(jg - NBUF), ssems[b]).wait()

                gen_chunk(j, bufs[b])
                pltpu.async_copy(bufs[b], out_rows(jg), ssems[b])
            return ()

        lax.fori_loop(0, NPART // NBUF, step, (), unroll=False)

    for b in range(NBUF):
        jg = NCHUNKS - NBUF + b
        pltpu.make_async_copy(bufs[b], out_rows(jg), ssems[b]).wait()


@jax.jit
def _lookup(idx, W_flat):
    mesh = plsc.VectorSubcoreMesh(core_axis_name="c", subcore_axis_name="s")
    f = pl.kernel(
        _sc_body,
        out_type=jax.ShapeDtypeStruct((B_TOTAL, D), jnp.float32),
        mesh=mesh,
        compiler_params=pltpu.CompilerParams(needs_layout_passes=False),
        scratch_types=[
            pltpu.VMEM((NPART, CHUNK), jnp.int32),
            pltpu.VMEM((VOCAB_PAD * D,), jnp.float32),
            pltpu.VMEM((CHUNK, D), jnp.float32),
            pltpu.VMEM((CHUNK, D), jnp.float32),
            pltpu.SemaphoreType.DMA,
            pltpu.SemaphoreType.DMA,
        ],
    )
    return f(idx, W_flat)


def kernel(x, W):
    idx = x[:, :, 1].astype(jnp.int32).reshape(NW, NSTAGE, NPART, CHUNK)
    W_pad = jnp.pad(W, ((0, VOCAB_PAD - W.shape[0]), (0, 0)))
    out = _lookup(idx, W_pad.reshape(VOCAB_PAD * D))
    return out.reshape(4096, 200, D)
